# Initial kernel scaffold; baseline (speedup 1.0000x reference)
import jax
import jax.numpy as jnp
from jax.experimental import pallas as pl


def kernel(x, edge_index, edge_attr, We1, be1, W1a, b1a, W1b, b1b, We2, be2, W2a, b2a, W2b, b2b):
    raise NotImplementedError("write your pallas kernel here")


# speedup vs baseline: 2.4685x; 2.4685x over previous
import functools

import jax
import jax.numpy as jnp
from jax import lax
from jax.experimental import pallas as pl
from jax.experimental.pallas import tpu as pltpu
from jax.experimental.pallas import tpu_sc as plsc

N = 10000
E = 320000
D = 128
DE = 16

NC = 2
NS = 16
NW = NC * NS
C = 80
EPW = E // NW
NCH = EPW // C
RPS = 624
RTAIL = N - NS * RPS

_mesh = plsc.VectorSubcoreMesh(core_axis_name="c", subcore_axis_name="s")


def _sc_message(h, src, dst, el, zz):
    @functools.partial(
        pl.kernel,
        mesh=_mesh,
        out_type=jax.ShapeDtypeStruct((NC, N, D), jnp.float32),
        scratch_types=[
            pltpu.VMEM_SHARED((N, D), jnp.float32),
            pltpu.VMEM((C,), jnp.int32),
            pltpu.VMEM((C,), jnp.int32),
            pltpu.VMEM((C, D), jnp.float32),
            pltpu.VMEM((C, D), jnp.float32),
            pltpu.SemaphoreType.DMA,
        ],
    )
    def k(h_hbm, src_hbm, dst_hbm, el_hbm, zz_hbm, out_hbm,
          acc, sidx, didx, elv, rows, sem):
        cid = lax.axis_index("c")
        sid = lax.axis_index("s")
        wid = sid * NC + cid
        pltpu.sync_copy(zz_hbm.at[pl.ds(sid * RPS, RPS)],
                        acc.at[pl.ds(sid * RPS, RPS)])

        @pl.when(sid == NS - 1)
        def _():
            pltpu.sync_copy(zz_hbm.at[pl.ds(NS * RPS, RTAIL)],
                            acc.at[pl.ds(NS * RPS, RTAIL)])

        plsc.subcore_barrier()
        base = wid * EPW

        def body(i, carry):
            off = base + i * C
            pltpu.sync_copy(src_hbm.at[pl.ds(off, C)], sidx)
            pltpu.sync_copy(dst_hbm.at[pl.ds(off, C)], didx)
            pltpu.sync_copy(el_hbm.at[pl.ds(off, C)], elv)
            pltpu.async_copy(h_hbm.at[sidx], rows, sem).wait()

            def rb(r, c2):
                for c8 in range(D // 16):
                    sl = pl.ds(c8 * 16, 16)
                    rows[r, sl] = jnp.maximum(rows[r, sl] + elv[r, sl], 0.0)
                return c2

            lax.fori_loop(0, C, rb, 0)
            pltpu.sync_copy(rows, acc.at[didx], add=True)
            return carry

        lax.fori_loop(0, NCH, body, 0)
        plsc.subcore_barrier()
        pltpu.sync_copy(acc.at[pl.ds(sid * RPS, RPS)],
                        out_hbm.at[cid, pl.ds(sid * RPS, RPS)])

        @pl.when(sid == NS - 1)
        def _():
            pltpu.sync_copy(acc.at[pl.ds(NS * RPS, RTAIL)],
                            out_hbm.at[cid, pl.ds(NS * RPS, RTAIL)])

    return k(h, src, dst, el, zz)


BE = 4000


def _edge_lin(ea, W, b):
    def k(ea_ref, w_ref, b_ref, o_ref):
        o_ref[...] = (
            jnp.dot(ea_ref[...], w_ref[...], preferred_element_type=jnp.float32)
            + b_ref[...]
        )

    return pl.pallas_call(
        k,
        grid=(E // BE,),
        in_specs=[
            pl.BlockSpec((BE, DE), lambda i: (i, 0)),
            pl.BlockSpec((DE, D), lambda i: (0, 0)),
            pl.BlockSpec((D,), lambda i: (0,)),
        ],
        out_specs=pl.BlockSpec((BE, D), lambda i: (i, 0)),
        out_shape=jax.ShapeDtypeStruct((E, D), jnp.float32),
    )(ea, W, b)


BN = 2000


def _combine(h, u0, u1, Wa, ba, Wb, bb):
    def k(h_ref, u0_ref, u1_ref, wa_ref, ba_ref, wb_ref, bb_ref, o_ref):
        z = h_ref[...] + u0_ref[...] + u1_ref[...]
        z = jnp.maximum(
            jnp.dot(z, wa_ref[...], preferred_element_type=jnp.float32)
            + ba_ref[...], 0.0)
        z = (jnp.dot(z, wb_ref[...], preferred_element_type=jnp.float32)
             + bb_ref[...])
        o_ref[...] = jnp.maximum(z, 0.0)

    blk = pl.BlockSpec((BN, D), lambda i: (i, 0))
    wblk = pl.BlockSpec((D, D), lambda i: (0, 0))
    bblk = pl.BlockSpec((D,), lambda i: (0,))
    return pl.pallas_call(
        k,
        grid=(N // BN,),
        in_specs=[blk, blk, blk, wblk, bblk, wblk, bblk],
        out_specs=blk,
        out_shape=jax.ShapeDtypeStruct((N, D), jnp.float32),
    )(h, u0, u1, Wa, ba, Wb, bb)


def kernel(x, edge_index, edge_attr,
           We1, be1, W1a, b1a, W1b, b1b,
           We2, be2, W2a, b2a, W2b, b2b):
    src = edge_index[0]
    dst = edge_index[1]
    zz = jnp.zeros((N, D), jnp.float32)
    el1 = _edge_lin(edge_attr, We1, be1)
    el2 = _edge_lin(edge_attr, We2, be2)
    u1 = _sc_message(x, src, dst, el1, zz)
    h1 = _combine(x, u1[0], u1[1], W1a, b1a, W1b, b1b)
    u2 = _sc_message(h1, src, dst, el2, zz)
    h2 = _combine(h1, u2[0], u2[1], W2a, b2a, W2b, b2b)
    return h2
